# baseline (speedup 1.0000x reference)
import jax
import jax.numpy as jnp
from jax import lax
from jax.experimental import pallas as pl
from jax.experimental.pallas import tpu as pltpu
from jax.experimental.pallas import tpu_sc as plsc

HID = 768
NC = 2
NS = 16
NW = NC * NS
B = 1024
S = 50
NBOX = 36
BATCH_PER_W = B // NW
VFEAT = 2048
EPS = 1e-12
PAD_S = 56


PIECES = (24, 32)
NPIECE = BATCH_PER_W * 2
TOK_PER_W = BATCH_PER_W * PAD_S


def _sc_gather_body(tok, wemb, out, idx_v, bufA, bufB, bufC, bufD,
                    gsA, gsB, gsC, gsD, wsA, wsB, wsC, wsD):
    c = lax.axis_index("c")
    s = lax.axis_index("s")
    wid = s * NC + c
    row0_w = wid * BATCH_PER_W

    bufs = (bufA, bufB, bufC, bufD)
    gsems = (gsA, gsB, gsC, gsD)
    wsems = (wsA, wsB, wsC, wsD)

    pltpu.sync_copy(tok.at[pl.ds(wid * TOK_PER_W, TOK_PER_W)], idx_v)

    def piece(p, slot):
        n = PIECES[slot % 2]
        off = (slot % 2) * PIECES[0]
        batch = row0_w + p // 2
        irel = (p // 2) * PAD_S + off
        return n, off, batch, irel

    def start_gather(p, slot):
        n, off, batch, irel = piece(p, slot)
        pltpu.async_copy(wemb.at[idx_v.at[pl.ds(irel, n)]], bufs[slot],
                         gsems[slot])

    def wait_gather(p, slot):
        n, off, batch, irel = piece(p, slot)
        pltpu.make_async_copy(wemb.at[idx_v.at[pl.ds(irel, n)]], bufs[slot],
                              gsems[slot]).wait()

    def start_write(p, slot):
        n, off, batch, irel = piece(p, slot)
        pltpu.async_copy(bufs[slot], out.at[batch, pl.ds(off, n)],
                         wsems[slot])

    def wait_write(p, slot):
        n, off, batch, irel = piece(p, slot)
        pltpu.make_async_copy(bufs[slot], out.at[batch, pl.ds(off, n)],
                              wsems[slot]).wait()

    for slot in range(4):
        start_gather(slot, slot)

    def loop_body(i, carry):
        p0 = 4 * i
        for slot in range(4):
            p = p0 + slot
            wait_gather(p, slot)
            start_write(p, slot)

            @pl.when(p + 4 < NPIECE)
            def _():
                wait_write(p, slot)
                start_gather(p + 4, slot)
        return carry

    lax.fori_loop(0, NPIECE // 4, loop_body, 0)
    for slot in range(4):
        wait_write(NPIECE - 4 + slot, slot)


def _sc_gather(tok_flat, word_emb):
    mesh = plsc.VectorSubcoreMesh(core_axis_name="c", subcore_axis_name="s")
    fn = pl.kernel(
        _sc_gather_body,
        mesh=mesh,
        compiler_params=pltpu.CompilerParams(needs_layout_passes=False),
        out_type=jax.ShapeDtypeStruct((B, PAD_S, HID), jnp.float32),
        scratch_types=[
            pltpu.VMEM((TOK_PER_W,), jnp.int32),
            pltpu.VMEM((PIECES[0], HID), jnp.float32),
            pltpu.VMEM((PIECES[1], HID), jnp.float32),
            pltpu.VMEM((PIECES[0], HID), jnp.float32),
            pltpu.VMEM((PIECES[1], HID), jnp.float32),
            pltpu.SemaphoreType.DMA,
            pltpu.SemaphoreType.DMA,
            pltpu.SemaphoreType.DMA,
            pltpu.SemaphoreType.DMA,
            pltpu.SemaphoreType.DMA,
            pltpu.SemaphoreType.DMA,
            pltpu.SemaphoreType.DMA,
            pltpu.SemaphoreType.DMA,
        ],
    )
    return fn(tok_flat, word_emb)


def _ln_tc(x, w, b):
    mu = jnp.mean(x, axis=-1, keepdims=True)
    d = x - mu
    var = jnp.mean(d * d, axis=-1, keepdims=True)
    return d * lax.rsqrt(var + jnp.float32(EPS)) * w + b


TBT = 16


def _tc_text_body(raw, bias, lnw, lnb, out):
    x = raw[...][:, :S, :] + bias[...]
    out[...] = _ln_tc(x, lnw[...], lnb[...])


def _tc_text(raw56, bias3, ln_w, ln_b):
    grid = B // TBT
    return pl.pallas_call(
        _tc_text_body,
        grid=(grid,),
        in_specs=[
            pl.BlockSpec((TBT, PAD_S, HID), lambda i: (i, 0, 0)),
            pl.BlockSpec((1, S, HID), lambda i: (0, 0, 0)),
            pl.BlockSpec((1, 1, HID), lambda i: (0, 0, 0)),
            pl.BlockSpec((1, 1, HID), lambda i: (0, 0, 0)),
        ],
        out_specs=pl.BlockSpec((TBT, S, HID), lambda i: (i, 0, 0)),
        out_shape=jax.ShapeDtypeStruct((B, S, HID), jnp.float32),
        compiler_params=pltpu.CompilerParams(
            dimension_semantics=("parallel",)),
    )(raw56, bias3, ln_w.reshape(1, 1, HID), ln_b.reshape(1, 1, HID))


TB = 16


def _tc_img_body(feat, loc, imgW, locW, typ, img_b, loc_b,
                 img_lnw, img_lnb, loc_lnw, loc_lnb, v_lnw, v_lnb, out):
    w = imgW[...]
    lw = locW[...]
    trow = typ[1:2, :]
    f = feat[...].reshape(TB * NBOX, VFEAT).astype(jnp.bfloat16)
    img = jnp.dot(f, w, preferred_element_type=jnp.float32)
    img = _ln_tc(img + img_b[...], img_lnw[...], img_lnb[...])
    l = jnp.dot(loc[...].reshape(TB * NBOX, 5), lw,
                preferred_element_type=jnp.float32)
    l = _ln_tc(l + loc_b[...], loc_lnw[...], loc_lnb[...])
    v = img + l + trow
    out[...] = _ln_tc(v, v_lnw[...], v_lnb[...]).reshape(TB, NBOX, HID)


def _tc_img(image_feat, image_loc, imgW_bf, loc_W, type_emb, img_b, loc_b,
            img_ln_w, img_ln_b, loc_ln_w, loc_ln_b, v_ln_w, v_ln_b):
    grid = B // TB
    row_spec = lambda i: (i, 0, 0)
    const_spec = lambda i: (0, 0)
    return pl.pallas_call(
        _tc_img_body,
        grid=(grid,),
        in_specs=[
            pl.BlockSpec((TB, NBOX, VFEAT), row_spec),
            pl.BlockSpec((TB, NBOX, 5), row_spec),
            pl.BlockSpec((VFEAT, HID), const_spec),
            pl.BlockSpec((5, HID), const_spec),
            pl.BlockSpec((2, HID), const_spec),
            pl.BlockSpec((1, HID), const_spec),
            pl.BlockSpec((1, HID), const_spec),
            pl.BlockSpec((1, HID), const_spec),
            pl.BlockSpec((1, HID), const_spec),
            pl.BlockSpec((1, HID), const_spec),
            pl.BlockSpec((1, HID), const_spec),
            pl.BlockSpec((1, HID), const_spec),
            pl.BlockSpec((1, HID), const_spec),
        ],
        out_specs=pl.BlockSpec((TB, NBOX, HID), row_spec),
        out_shape=jax.ShapeDtypeStruct((B, NBOX, HID), jnp.float32),
        compiler_params=pltpu.CompilerParams(
            dimension_semantics=("parallel",)),
    )(image_feat, image_loc, imgW_bf, loc_W, type_emb, img_b, loc_b,
      img_ln_w, img_ln_b, loc_ln_w, loc_ln_b, v_ln_w, v_ln_b)


def kernel(token_ids, image_feat, image_loc, word_emb, pos_emb, type_emb,
           ln_w, ln_b, img_W, img_b, loc_W, loc_b,
           img_ln_w, img_ln_b, loc_ln_w, loc_ln_b, v_ln_w, v_ln_b):
    tok_pad = jnp.concatenate(
        [token_ids.astype(jnp.int32), jnp.zeros((B, PAD_S - S), jnp.int32)],
        axis=1)
    raw56 = _sc_gather(tok_pad.reshape(B * PAD_S), word_emb)

    r2 = lambda a: a.reshape(1, HID)
    v_emb = _tc_img(image_feat, image_loc, img_W.astype(jnp.bfloat16), loc_W,
                    type_emb, r2(img_b), r2(loc_b), r2(img_ln_w), r2(img_ln_b),
                    r2(loc_ln_w), r2(loc_ln_b), r2(v_ln_w), r2(v_ln_b))

    bias3 = (pos_emb[:S] + type_emb[0]).reshape(1, S, HID)
    emb = _tc_text(raw56, bias3, ln_w, ln_b)

    return (emb, v_emb)


# speedup vs baseline: 1.0009x; 1.0009x over previous
import jax
import jax.numpy as jnp
from jax import lax
from jax.experimental import pallas as pl
from jax.experimental.pallas import tpu as pltpu
from jax.experimental.pallas import tpu_sc as plsc

HID = 768
NC = 2
NS = 16
NW = NC * NS
B = 1024
S = 50
NBOX = 36
BATCH_PER_W = B // NW
VFEAT = 2048
EPS = 1e-12
PAD_S = 56


PIECES = (24, 32)
NPIECE = BATCH_PER_W * 2
TOK_PER_W = BATCH_PER_W * PAD_S


def _sc_gather_body(tok, wemb, out, idx_v, bufA, bufB, bufC, bufD,
                    gsA, gsB, gsC, gsD, wsA, wsB, wsC, wsD):
    c = lax.axis_index("c")
    s = lax.axis_index("s")
    wid = s * NC + c
    row0_w = wid * BATCH_PER_W

    bufs = (bufA, bufB, bufC, bufD)
    gsems = (gsA, gsB, gsC, gsD)
    wsems = (wsA, wsB, wsC, wsD)

    pltpu.sync_copy(tok.at[pl.ds(wid * TOK_PER_W, TOK_PER_W)], idx_v)

    def piece(p, slot):
        n = PIECES[slot % 2]
        off = (slot % 2) * PIECES[0]
        batch = row0_w + p // 2
        irel = (p // 2) * PAD_S + off
        return n, off, batch, irel

    def start_gather(p, slot):
        n, off, batch, irel = piece(p, slot)
        pltpu.async_copy(wemb.at[idx_v.at[pl.ds(irel, n)]], bufs[slot],
                         gsems[slot])

    def wait_gather(p, slot):
        n, off, batch, irel = piece(p, slot)
        pltpu.make_async_copy(wemb.at[idx_v.at[pl.ds(irel, n)]], bufs[slot],
                              gsems[slot]).wait()

    def start_write(p, slot):
        n, off, batch, irel = piece(p, slot)
        pltpu.async_copy(bufs[slot], out.at[batch, pl.ds(off, n)],
                         wsems[slot])

    def wait_write(p, slot):
        n, off, batch, irel = piece(p, slot)
        pltpu.make_async_copy(bufs[slot], out.at[batch, pl.ds(off, n)],
                              wsems[slot]).wait()

    for slot in range(4):
        start_gather(slot, slot)

    def loop_body(i, carry):
        p0 = 4 * i
        for slot in range(4):
            p = p0 + slot
            wait_gather(p, slot)
            start_write(p, slot)

            @pl.when(p + 4 < NPIECE)
            def _():
                wait_write(p, slot)
                start_gather(p + 4, slot)
        return carry

    lax.fori_loop(0, NPIECE // 4, loop_body, 0)
    for slot in range(4):
        wait_write(NPIECE - 4 + slot, slot)


def _sc_gather(tok_flat, word_emb):
    mesh = plsc.VectorSubcoreMesh(core_axis_name="c", subcore_axis_name="s")
    fn = pl.kernel(
        _sc_gather_body,
        mesh=mesh,
        compiler_params=pltpu.CompilerParams(needs_layout_passes=False),
        cost_estimate=pl.CostEstimate(
            flops=0,
            bytes_accessed=2 * B * PAD_S * HID * 4,
            transcendentals=0),
        out_type=jax.ShapeDtypeStruct((B, PAD_S, HID), jnp.float32),
        scratch_types=[
            pltpu.VMEM((TOK_PER_W,), jnp.int32),
            pltpu.VMEM((PIECES[0], HID), jnp.float32),
            pltpu.VMEM((PIECES[1], HID), jnp.float32),
            pltpu.VMEM((PIECES[0], HID), jnp.float32),
            pltpu.VMEM((PIECES[1], HID), jnp.float32),
            pltpu.SemaphoreType.DMA,
            pltpu.SemaphoreType.DMA,
            pltpu.SemaphoreType.DMA,
            pltpu.SemaphoreType.DMA,
            pltpu.SemaphoreType.DMA,
            pltpu.SemaphoreType.DMA,
            pltpu.SemaphoreType.DMA,
            pltpu.SemaphoreType.DMA,
        ],
    )
    return fn(tok_flat, word_emb)


def _ln_tc(x, w, b):
    mu = jnp.mean(x, axis=-1, keepdims=True)
    d = x - mu
    var = jnp.mean(d * d, axis=-1, keepdims=True)
    return d * lax.rsqrt(var + jnp.float32(EPS)) * w + b


TBT = 16


def _tc_text_body(raw, bias, lnw, lnb, out):
    x = raw[...][:, :S, :] + bias[...]
    out[...] = _ln_tc(x, lnw[...], lnb[...])


def _tc_text(raw56, bias3, ln_w, ln_b):
    grid = B // TBT
    return pl.pallas_call(
        _tc_text_body,
        grid=(grid,),
        in_specs=[
            pl.BlockSpec((TBT, PAD_S, HID), lambda i: (i, 0, 0)),
            pl.BlockSpec((1, S, HID), lambda i: (0, 0, 0)),
            pl.BlockSpec((1, 1, HID), lambda i: (0, 0, 0)),
            pl.BlockSpec((1, 1, HID), lambda i: (0, 0, 0)),
        ],
        out_specs=pl.BlockSpec((TBT, S, HID), lambda i: (i, 0, 0)),
        out_shape=jax.ShapeDtypeStruct((B, S, HID), jnp.float32),
        compiler_params=pltpu.CompilerParams(
            dimension_semantics=("parallel",)),
    )(raw56, bias3, ln_w.reshape(1, 1, HID), ln_b.reshape(1, 1, HID))


TB = 16


def _tc_img_body(feat, loc, imgW, locW, typ, img_b, loc_b,
                 img_lnw, img_lnb, loc_lnw, loc_lnb, v_lnw, v_lnb, out):
    w = imgW[...]
    lw = locW[...]
    trow = typ[1:2, :]
    f = feat[...].reshape(TB * NBOX, VFEAT).astype(jnp.bfloat16)
    img = jnp.dot(f, w, preferred_element_type=jnp.float32)
    img = _ln_tc(img + img_b[...], img_lnw[...], img_lnb[...])
    l = jnp.dot(loc[...].reshape(TB * NBOX, 5), lw,
                preferred_element_type=jnp.float32)
    l = _ln_tc(l + loc_b[...], loc_lnw[...], loc_lnb[...])
    v = img + l + trow
    out[...] = _ln_tc(v, v_lnw[...], v_lnb[...]).reshape(TB, NBOX, HID)


def _tc_img(image_feat, image_loc, imgW_bf, loc_W, type_emb, img_b, loc_b,
            img_ln_w, img_ln_b, loc_ln_w, loc_ln_b, v_ln_w, v_ln_b):
    grid = B // TB
    row_spec = lambda i: (i, 0, 0)
    const_spec = lambda i: (0, 0)
    return pl.pallas_call(
        _tc_img_body,
        grid=(grid,),
        in_specs=[
            pl.BlockSpec((TB, NBOX, VFEAT), row_spec),
            pl.BlockSpec((TB, NBOX, 5), row_spec),
            pl.BlockSpec((VFEAT, HID), const_spec),
            pl.BlockSpec((5, HID), const_spec),
            pl.BlockSpec((2, HID), const_spec),
            pl.BlockSpec((1, HID), const_spec),
            pl.BlockSpec((1, HID), const_spec),
            pl.BlockSpec((1, HID), const_spec),
            pl.BlockSpec((1, HID), const_spec),
            pl.BlockSpec((1, HID), const_spec),
            pl.BlockSpec((1, HID), const_spec),
            pl.BlockSpec((1, HID), const_spec),
            pl.BlockSpec((1, HID), const_spec),
        ],
        out_specs=pl.BlockSpec((TB, NBOX, HID), row_spec),
        out_shape=jax.ShapeDtypeStruct((B, NBOX, HID), jnp.float32),
        compiler_params=pltpu.CompilerParams(
            dimension_semantics=("parallel",)),
    )(image_feat, image_loc, imgW_bf, loc_W, type_emb, img_b, loc_b,
      img_ln_w, img_ln_b, loc_ln_w, loc_ln_b, v_ln_w, v_ln_b)


def kernel(token_ids, image_feat, image_loc, word_emb, pos_emb, type_emb,
           ln_w, ln_b, img_W, img_b, loc_W, loc_b,
           img_ln_w, img_ln_b, loc_ln_w, loc_ln_b, v_ln_w, v_ln_b):
    tok_pad = jnp.concatenate(
        [token_ids.astype(jnp.int32), jnp.zeros((B, PAD_S - S), jnp.int32)],
        axis=1)
    raw56 = _sc_gather(tok_pad.reshape(B * PAD_S), word_emb)

    r2 = lambda a: a.reshape(1, HID)
    v_emb = _tc_img(image_feat, image_loc, img_W.astype(jnp.bfloat16), loc_W,
                    type_emb, r2(img_b), r2(loc_b), r2(img_ln_w), r2(img_ln_b),
                    r2(loc_ln_w), r2(loc_ln_b), r2(v_ln_w), r2(v_ln_b))

    bias3 = (pos_emb[:S] + type_emb[0]).reshape(1, S, HID)
    emb = _tc_text(raw56, bias3, ln_w, ln_b)

    return (emb, v_emb)
